# baseline (speedup 1.0000x reference)
import functools

import jax
import jax.numpy as jnp
from jax.experimental import pallas as pl
from jax.experimental.pallas import tpu as pltpu
from jax.experimental.pallas import tpu_sc as plsc

ALPHA = 0.1
K = 10
NS = 16
EB = 128
NCH = 160
RSUB = 128
NPAD = 10240
CH = 32

_SC_PARAMS = pltpu.CompilerParams(use_tc_tiling_on_sc=False,
                                  needs_layout_passes=False)
_MESH = dict(core_axis_name="c", subcore_axis_name="s")


def _tc_mlp(x, W1, b1, W2, b2):
    n = x.shape[0]
    c = W2.shape[1]
    blk = 1280
    nb = n // blk

    def body(x_ref, w1_ref, b1_ref, w2_ref, b2_ref, hs_ref, ah_ref):
        j = pl.program_id(0)
        h1 = jnp.maximum(
            jnp.dot(x_ref[...], w1_ref[...], preferred_element_type=jnp.float32)
            + b1_ref[...], 0.0)
        h = jnp.dot(h1, w2_ref[...], preferred_element_type=jnp.float32) + b2_ref[...]
        hs_ref[...] = jnp.where(j == 0, h[:, :CH], h[:, CH:])
        ah_ref[...] = ALPHA * h

    f = jnp.float32
    return pl.pallas_call(
        body,
        grid=(2, nb),
        in_specs=[
            pl.BlockSpec((blk, x.shape[1]), lambda j, i: (i, 0)),
            pl.BlockSpec(W1.shape, lambda j, i: (0, 0)),
            pl.BlockSpec((1, b1.shape[0]), lambda j, i: (0, 0)),
            pl.BlockSpec(W2.shape, lambda j, i: (0, 0)),
            pl.BlockSpec((1, b2.shape[0]), lambda j, i: (0, 0)),
        ],
        out_specs=[pl.BlockSpec((blk, CH), lambda j, i: (j * nb + i, 0)),
                   pl.BlockSpec((blk, c), lambda j, i: (i, 0))],
        out_shape=(jax.ShapeDtypeStruct((2 * n, CH), f),
                   jax.ShapeDtypeStruct((n, c), f)),
    )(x, W1, b1.reshape(1, -1), W2, b2.reshape(1, -1))


def _rsqrt16(d):
    i = plsc.bitcast(d, jnp.int32)
    i = jnp.int32(0x5F3759DF) - jnp.right_shift(i, jnp.int32(1))
    y = plsc.bitcast(i, jnp.float32)
    for _ in range(3):
        y = y * (1.5 - 0.5 * d * y * y)
    return y


def _sc_fused(hs, src_flat, dst_flat):
    rpt = NPAD // NS
    nsub = rpt // RSUB
    ept = NCH * EB
    fo = jax.ShapeDtypeStruct((2 * NPAD, CH), jnp.float32)

    @functools.partial(
        pl.kernel,
        out_type=(fo, fo, fo, fo),
        mesh=plsc.VectorSubcoreMesh(**_MESH),
        compiler_params=_SC_PARAMS,
        scratch_types=[
            pltpu.VMEM_SHARED((NPAD, CH), jnp.float32),
            pltpu.VMEM_SHARED((NPAD, CH), jnp.float32),
            pltpu.VMEM((NCH * EB,), jnp.int32),
            pltpu.VMEM((NCH * EB,), jnp.int32),
            pltpu.VMEM((RSUB, CH), jnp.float32),
            pltpu.VMEM((RSUB, CH), jnp.float32),
            pltpu.VMEM((RSUB, CH), jnp.float32),
            pltpu.VMEM((RSUB, CH), jnp.float32),
            pltpu.VMEM((2 * EB, CH), jnp.float32),
            pltpu.VMEM((2 * EB, CH), jnp.float32),
            pltpu.VMEM((RSUB, CH), jnp.float32),
            pltpu.VMEM((RSUB, CH), jnp.float32),
            pltpu.SemaphoreType.DMA,
            pltpu.SemaphoreType.DMA,
            pltpu.SemaphoreType.DMA,
            pltpu.SemaphoreType.DMA,
            pltpu.SemaphoreType.DMA,
            pltpu.SemaphoreType.DMA,
        ],
    )
    def fused_kernel(hs_hbm, src_hbm, dst_hbm,
                     sout_hbm, u_hbm, ht_hbm, df_hbm,
                     zt, agg, sidx, didx, ubuf_a, hbuf_a, ubuf_b, hbuf_b,
                     rows_a, rows_b, abuf_a, abuf_b,
                     gsa, gsb, ssa, ssb, lsa, lsb):
        core = jax.lax.axis_index("c")
        tile = jax.lax.axis_index("s")
        r0 = tile * rpt
        h0 = core * NPAD + r0
        GB = 2 * EB

        pltpu.async_copy(src_hbm.at[pl.ds(tile * ept, ept)], sidx, lsa)
        pltpu.async_copy(dst_hbm.at[pl.ds(tile * ept, ept)], didx, lsb)

        zeros16 = jnp.zeros((16,), jnp.float32)
        ones16 = jnp.ones((16,), jnp.float32)

        @pl.loop(0, RSUB)
        def _(j):
            for cc in range(CH // 16):
                abuf_a[j, pl.ds(cc * 16, 16)] = zeros16

        @pl.loop(0, GB)
        def _(j):
            for cc in range(CH // 16):
                rows_a[j, pl.ds(cc * 16, 16)] = ones16

        for sub in range(nsub):
            pltpu.sync_copy(abuf_a, agg.at[pl.ds(r0 + sub * RSUB, RSUB)])
        pltpu.make_async_copy(src_hbm.at[pl.ds(tile * ept, ept)], sidx,
                              lsa).wait()
        pltpu.make_async_copy(dst_hbm.at[pl.ds(tile * ept, ept)], didx,
                              lsb).wait()
        plsc.subcore_barrier()

        @pl.loop(0, ept // GB)
        def _(i):
            pltpu.async_copy(rows_a, agg.at[didx.at[pl.ds(i * GB, GB)]], ssa,
                             add=True)

        @pl.loop(0, ept // GB)
        def _(i):
            pltpu.make_async_copy(rows_a, agg.at[didx.at[pl.ds(0, GB)]],
                                  ssa).wait()

        plsc.subcore_barrier()

        for sub in range(nsub):
            ssp = pl.ds(r0 + sub * RSUB, RSUB)
            shb = pl.ds(h0 + sub * RSUB, RSUB)
            pltpu.sync_copy(agg.at[ssp], abuf_a)
            pltpu.sync_copy(hs_hbm.at[shb], hbuf_a)

            @pl.loop(0, RSUB)
            def _(j):
                d = abuf_a[j, pl.ds(0, 16)] + 1.0
                y = _rsqrt16(d)
                u = (1.0 - ALPHA) * y * y
                dd = (1.0 - ALPHA) * y
                for cc in range(CH // 16):
                    csl = pl.ds(cc * 16, 16)
                    ubuf_a[j, csl] = u
                    abuf_b[j, csl] = dd
                    hv = hbuf_a[j, csl]
                    hbuf_b[j, csl] = ALPHA * y * hv
                    hbuf_a[j, csl] = y * hv

            pltpu.sync_copy(ubuf_a, u_hbm.at[shb])
            pltpu.sync_copy(abuf_b, df_hbm.at[shb])
            pltpu.sync_copy(hbuf_b, ht_hbm.at[shb])
            pltpu.sync_copy(hbuf_a, zt.at[ssp])
            pltpu.sync_copy(hbuf_a, agg.at[ssp])
        plsc.subcore_barrier()

        def g_start(i, buf, sem):
            pltpu.async_copy(zt.at[sidx.at[pl.ds(i * GB, GB)]], buf, sem)

        def g_wait(buf, sem):
            pltpu.make_async_copy(zt.at[sidx.at[pl.ds(0, GB)]], buf, sem).wait()

        def s_start(i, buf, sem):
            pltpu.async_copy(buf, agg.at[didx.at[pl.ds(i * GB, GB)]], sem,
                             add=True)

        def s_wait(buf, sem):
            pltpu.make_async_copy(buf, agg.at[didx.at[pl.ds(0, GB)]], sem).wait()

        npairs = NCH // 4

        def edge_pass():
            g_start(0, rows_a, gsa)

            @pl.loop(0, npairs)
            def _(p):
                i0 = 2 * p
                g_wait(rows_a, gsa)

                @pl.when(p > 0)
                def _():
                    s_wait(rows_b, ssb)

                g_start(i0 + 1, rows_b, gsb)
                s_start(i0, rows_a, ssa)
                g_wait(rows_b, gsb)
                s_wait(rows_a, ssa)

                @pl.when(p < npairs - 1)
                def _():
                    g_start(i0 + 2, rows_a, gsa)

                s_start(i0 + 1, rows_b, ssb)

            s_wait(rows_b, ssb)

        @pl.loop(0, K - 1)
        def _(k):
            edge_pass()
            plsc.subcore_barrier()
            bufs = ((abuf_a, ubuf_a, hbuf_a, lsa), (abuf_b, ubuf_b, hbuf_b, lsb))

            def ew_load(sub, bs):
                ab, ub, hb, sem = bs
                pltpu.async_copy(u_hbm.at[pl.ds(h0 + sub * RSUB, RSUB)], ub, sem)
                pltpu.async_copy(ht_hbm.at[pl.ds(h0 + sub * RSUB, RSUB)], hb, sem)

            def ew_wait(bs):
                ab, ub, hb, sem = bs
                pltpu.make_async_copy(u_hbm.at[pl.ds(h0, RSUB)], ub, sem).wait()
                pltpu.make_async_copy(ht_hbm.at[pl.ds(h0, RSUB)], hb, sem).wait()

            ew_load(0, bufs[0])
            for sub in range(nsub):
                bs = bufs[sub % 2]
                ab, ub, hb, _ = bs
                pltpu.sync_copy(agg.at[pl.ds(r0 + sub * RSUB, RSUB)], ab)
                ew_wait(bs)
                if sub < nsub - 1:
                    ew_load(sub + 1, bufs[(sub + 1) % 2])

                @pl.loop(0, RSUB)
                def _(j):
                    for cc in range(CH // 16):
                        csl = pl.ds(cc * 16, 16)
                        ab[j, csl] = ub[j, csl] * ab[j, csl] + hb[j, csl]

                ssp = pl.ds(r0 + sub * RSUB, RSUB)
                pltpu.sync_copy(ab, zt.at[ssp])
                pltpu.sync_copy(ab, agg.at[ssp])
            plsc.subcore_barrier()

        edge_pass()
        plsc.subcore_barrier()
        for sub in range(nsub):
            shb = pl.ds(h0 + sub * RSUB, RSUB)
            pltpu.sync_copy(agg.at[pl.ds(r0 + sub * RSUB, RSUB)], abuf_a)
            pltpu.sync_copy(df_hbm.at[shb], ubuf_a)

            @pl.loop(0, RSUB)
            def _(j):
                for cc in range(CH // 16):
                    csl = pl.ds(cc * 16, 16)
                    abuf_a[j, csl] = ubuf_a[j, csl] * abuf_a[j, csl]

            pltpu.sync_copy(abuf_a, sout_hbm.at[shb])

    return fused_kernel(hs, src_flat, dst_flat)[0]


def _tc_finish(sf, ah64, n):
    c = ah64.shape[1]
    blk = 2000
    s3 = sf.reshape(2, NPAD, CH)

    def body(slo_ref, shi_ref, ah_ref, o_ref):
        s = jnp.concatenate([slo_ref[...][0], shi_ref[...][0]], axis=1)
        z = s + ah_ref[...]
        m = jnp.max(z, axis=1, keepdims=True)
        lse = jnp.log(jnp.sum(jnp.exp(z - m), axis=1, keepdims=True)) + m
        o_ref[...] = z - lse

    return pl.pallas_call(
        body,
        grid=(n // blk,),
        in_specs=[pl.BlockSpec((1, blk, CH), lambda i: (0, i, 0)),
                  pl.BlockSpec((1, blk, CH), lambda i: (1, i, 0)),
                  pl.BlockSpec((blk, c), lambda i: (i, 0))],
        out_specs=pl.BlockSpec((blk, c), lambda i: (i, 0)),
        out_shape=jax.ShapeDtypeStruct((n, c), jnp.float32),
    )(s3, s3, ah64)


def kernel(x, edge_index, W1, b1, W2, b2):
    n = x.shape[0]
    e = edge_index.shape[1]
    epad = NS * NCH * EB
    xp = jnp.pad(x, ((0, NPAD - n), (0, 0)))
    pad = jnp.full((epad - e,), NPAD - 1, jnp.int32)
    srcf = jnp.concatenate([edge_index[0], pad])
    dstf = jnp.concatenate([edge_index[1], pad])
    hs, ah64 = _tc_mlp(xp, W1, b1, W2, b2)
    sf = _sc_fused(hs, srcf, dstf)
    return _tc_finish(sf, ah64, n)
